# baseline (speedup 1.0000x reference)
import functools

import jax
import jax.numpy as jnp
from jax import lax
from jax.experimental import pallas as pl
from jax.experimental.pallas import tpu as pltpu
from jax.experimental.pallas import tpu_sc as plsc

NC = 2
NS = 16
LANES = 16
NW = NC * NS

B = 16384
L = 20
D = 32
CH = 64
ROWS_PER_W = B // NW
NCH = ROWS_PER_W // CH
GSTEP = 128
NGS = CH * L // GSTEP
TT4_ROWS = 25008


def _body(tid_hbm, kid_hbm, tt4_hbm, ktab_hbm, out_hbm,
          tidx, t4idx, kidx, tbuf4, kbuf, obuf, sbuf, nbuf, t0buf,
          sg0, sg1, so0, so1):
    wid = lax.axis_index("s") * NC + lax.axis_index("c")
    base0 = wid * ROWS_PER_W
    sem_g = (sg0, sg1)
    sem_o = (so0, so1)

    pltpu.sync_copy(ktab_hbm.at[pl.ds(0, 1)], t0buf)
    t0a = t0buf[0, pl.ds(0, LANES)]
    t0b = t0buf[0, pl.ds(LANES, LANES)]
    lanes = lax.iota(jnp.int32, 16)

    def fire(b, base):
        ti4 = t4idx.at[pl.ds(b * CH, CH)]
        pltpu.sync_copy(tid_hbm.at[pl.ds(base, CH)], tidx.at[pl.ds(b * CH, CH)])
        pltpu.sync_copy(kid_hbm.at[pl.ds(base * L, CH * L)],
                        kidx.at[pl.ds(b * CH * L, CH * L)])
        for g in range(CH // LANES):
            o = b * CH + g * LANES
            t4idx[pl.ds(o, LANES)] = tidx[pl.ds(o, LANES)] >> 2
        pltpu.async_copy(tt4_hbm.at[ti4], tbuf4.at[pl.ds(b * CH, CH)],
                         sem_g[b])
        for p in range(NGS):
            o = b * CH * L + p * GSTEP
            pltpu.async_copy(ktab_hbm.at[kidx.at[pl.ds(o, GSTEP)]],
                             kbuf.at[pl.ds(o, GSTEP)], sem_g[b])

    def drain_gathers(b):
        ti4 = t4idx.at[pl.ds(b * CH, CH)]
        pltpu.make_async_copy(tt4_hbm.at[ti4],
                              tbuf4.at[pl.ds(b * CH, CH)], sem_g[b]).wait()
        for p in range(NGS):
            o = b * CH * L + p * GSTEP
            pltpu.make_async_copy(ktab_hbm.at[kidx.at[pl.ds(o, GSTEP)]],
                                  kbuf.at[pl.ds(o, GSTEP)], sem_g[b]).wait()

    def out_copy(b, base):
        return pltpu.make_async_copy(
            obuf.at[:, pl.ds(b * CH, CH)],
            out_hbm.at[:, pl.ds(base, CH)], sem_o[b])

    def compute(b, base):
        kb = b * CH * L
        for g in range(CH // LANES):
            acc = jnp.zeros((LANES,), jnp.int32)
            for j in range(L):
                ids = plsc.load_gather(kidx, [lanes * L + (kb + g * LANES * L + j)])
                acc = acc + jnp.where(ids != 0, 1, 0)
            nf = acc.astype(jnp.float32)
            bo = b * CH + g * LANES
            sbuf[pl.ds(bo, LANES)] = 1.0 / jnp.maximum(nf, 1.0)
            nbuf[pl.ds(bo, LANES)] = jnp.float32(L) - nf

        for g in range(CH // LANES):
            bo = b * CH + g * LANES
            rows_idx = bo + lanes
            m4 = (tidx[pl.ds(bo, LANES)] & 3) * D
            for d in range(D):
                v = plsc.load_gather(tbuf4, [rows_idx, m4 + d])
                obuf[d, pl.ds(bo, LANES)] = v

        def row_body(i, carry):
            r0 = kb + i * L
            acc0 = kbuf[r0, pl.ds(0, LANES)]
            acc1 = kbuf[r0, pl.ds(LANES, LANES)]
            for j in range(1, L):
                acc0 = acc0 + kbuf[r0 + j, pl.ds(0, LANES)]
                acc1 = acc1 + kbuf[r0 + j, pl.ds(LANES, LANES)]
            col = lanes * 0 + (b * CH + i)
            plsc.store_scatter(obuf, [2 * LANES + lanes, col], acc0)
            plsc.store_scatter(obuf, [3 * LANES + lanes, col], acc1)
            return carry

        lax.fori_loop(0, CH, row_body, 0, unroll=False)

        for g in range(CH // LANES):
            bo = b * CH + g * LANES
            sv = sbuf[pl.ds(bo, LANES)]
            n0v = nbuf[pl.ds(bo, LANES)]
            for d in range(D):
                t0d = t0a[d] if d < LANES else t0b[d - LANES]
                v = obuf[D + d, pl.ds(bo, LANES)]
                obuf[D + d, pl.ds(bo, LANES)] = (v - n0v * t0d) * sv

    fire(0, base0)

    def pair_body(k, carry):
        c0 = 2 * k
        fire(1, base0 + (c0 + 1) * CH)
        drain_gathers(0)

        @pl.when(k > 0)
        def _():
            out_copy(0, base0 + (c0 - 2) * CH).wait()

        compute(0, base0 + c0 * CH)
        out_copy(0, base0 + c0 * CH).start()

        @pl.when(c0 + 2 < NCH)
        def _():
            fire(0, base0 + (c0 + 2) * CH)

        drain_gathers(1)

        @pl.when(k > 0)
        def _():
            out_copy(1, base0 + (c0 - 1) * CH).wait()

        compute(1, base0 + (c0 + 1) * CH)
        out_copy(1, base0 + (c0 + 1) * CH).start()
        return carry

    lax.fori_loop(0, NCH // 2, pair_body, 0, unroll=False)
    out_copy(0, base0 + (NCH - 2) * CH).wait()
    out_copy(1, base0 + (NCH - 1) * CH).wait()


@jax.jit
def _run(title_ids, tok_flat, tt4, token_table):
    mesh = plsc.VectorSubcoreMesh(
        core_axis_name="c", subcore_axis_name="s",
        num_cores=NC, num_subcores=NS)
    f = pl.kernel(
        _body,
        out_type=jax.ShapeDtypeStruct((2 * D, B), jnp.float32),
        mesh=mesh,
        compiler_params=pltpu.CompilerParams(
            needs_layout_passes=False, use_tc_tiling_on_sc=False),
        scratch_types=[
            pltpu.VMEM((2 * CH,), jnp.int32),
            pltpu.VMEM((2 * CH,), jnp.int32),
            pltpu.VMEM((2 * CH * L,), jnp.int32),
            pltpu.VMEM((2 * CH, 128), jnp.float32),
            pltpu.VMEM((2 * CH * L, D), jnp.float32),
            pltpu.VMEM((2 * D, 2 * CH), jnp.float32),
            pltpu.VMEM((2 * CH,), jnp.float32),
            pltpu.VMEM((2 * CH,), jnp.float32),
            pltpu.VMEM((1, D), jnp.float32),
            pltpu.SemaphoreType.DMA,
            pltpu.SemaphoreType.DMA,
            pltpu.SemaphoreType.DMA,
            pltpu.SemaphoreType.DMA,
        ],
    )
    return f(title_ids, tok_flat, tt4, token_table)


def kernel(title_ids, token_ids, title_table, token_table):
    tok_flat = token_ids.reshape(B * L)
    tt4 = jnp.pad(title_table, ((0, 31), (0, 0))).reshape(TT4_ROWS, 128)
    out_t = _run(title_ids, tok_flat, tt4, token_table)
    return out_t.T


# speedup vs baseline: 1.5451x; 1.5451x over previous
import functools

import jax
import jax.numpy as jnp
from jax import lax
from jax.experimental import pallas as pl
from jax.experimental.pallas import tpu as pltpu
from jax.experimental.pallas import tpu_sc as plsc

NC = 2
NS = 16
LANES = 16
NW = NC * NS

B = 16384
L = 20
D = 32
CH = 64
ROWS_PER_W = B // NW
NCH = ROWS_PER_W // CH
GSTEP = 128
NGS = CH * L // GSTEP
NT = 100001
TW = CH * D
NTS = TW // GSTEP


def _body(tid_hbm, kid_hbm, ttf_hbm, ktab_hbm, out_hbm,
          tidx, tgidx, kidx, tbufT, kbuf, obuf, sbuf, nbuf, t0buf,
          sg0, sg1, so0, so1):
    wid = lax.axis_index("s") * NC + lax.axis_index("c")
    base0 = wid * ROWS_PER_W
    sem_g = (sg0, sg1)
    sem_o = (so0, so1)

    pltpu.sync_copy(ktab_hbm.at[pl.ds(0, 1)], t0buf)
    t0a = t0buf[0, pl.ds(0, LANES)]
    t0b = t0buf[0, pl.ds(LANES, LANES)]
    lanes = lax.iota(jnp.int32, 16)

    def fire(b, base):
        pltpu.sync_copy(tid_hbm.at[pl.ds(base, CH)], tidx.at[pl.ds(b * CH, CH)])
        pltpu.sync_copy(kid_hbm.at[pl.ds(base * L, CH * L)],
                        kidx.at[pl.ds(b * CH * L, CH * L)])
        for g in range(CH // LANES):
            v = tidx[pl.ds(b * CH + g * LANES, LANES)]
            for d in range(D):
                tgidx[pl.ds(b * TW + d * CH + g * LANES, LANES)] = v + d * NT
        for p in range(NTS):
            o = b * TW + p * GSTEP
            pltpu.async_copy(ttf_hbm.at[tgidx.at[pl.ds(o, GSTEP)]],
                             tbufT.at[pl.ds(o, GSTEP)], sem_g[b])
        for p in range(NGS):
            o = b * CH * L + p * GSTEP
            pltpu.async_copy(ktab_hbm.at[kidx.at[pl.ds(o, GSTEP)]],
                             kbuf.at[pl.ds(o, GSTEP)], sem_g[b])

    def drain_gathers(b):
        for p in range(NTS):
            o = b * TW + p * GSTEP
            pltpu.make_async_copy(ttf_hbm.at[tgidx.at[pl.ds(o, GSTEP)]],
                                  tbufT.at[pl.ds(o, GSTEP)], sem_g[b]).wait()
        for p in range(NGS):
            o = b * CH * L + p * GSTEP
            pltpu.make_async_copy(ktab_hbm.at[kidx.at[pl.ds(o, GSTEP)]],
                                  kbuf.at[pl.ds(o, GSTEP)], sem_g[b]).wait()

    def out_copy(b, base):
        return pltpu.make_async_copy(
            obuf.at[:, pl.ds(b * CH, CH)],
            out_hbm.at[:, pl.ds(base, CH)], sem_o[b])

    def compute(b, base):
        kb = b * CH * L
        for g in range(CH // LANES):
            acc = jnp.zeros((LANES,), jnp.int32)
            for j in range(L):
                ids = plsc.load_gather(kidx, [lanes * L + (kb + g * LANES * L + j)])
                acc = acc + jnp.where(ids != 0, 1, 0)
            nf = acc.astype(jnp.float32)
            bo = b * CH + g * LANES
            sbuf[pl.ds(bo, LANES)] = 1.0 / jnp.maximum(nf, 1.0)
            nbuf[pl.ds(bo, LANES)] = jnp.float32(L) - nf

        for g in range(CH // LANES):
            bo = b * CH + g * LANES
            for d in range(D):
                obuf[d, pl.ds(bo, LANES)] = tbufT[pl.ds(b * TW + d * CH
                                                        + g * LANES, LANES)]

        def row_body(i, carry):
            r0 = kb + i * L
            acc0 = kbuf[r0, pl.ds(0, LANES)]
            acc1 = kbuf[r0, pl.ds(LANES, LANES)]
            for j in range(1, L):
                acc0 = acc0 + kbuf[r0 + j, pl.ds(0, LANES)]
                acc1 = acc1 + kbuf[r0 + j, pl.ds(LANES, LANES)]
            col = lanes * 0 + (b * CH + i)
            plsc.store_scatter(obuf, [2 * LANES + lanes, col], acc0)
            plsc.store_scatter(obuf, [3 * LANES + lanes, col], acc1)
            return carry

        lax.fori_loop(0, CH, row_body, 0, unroll=False)

        for g in range(CH // LANES):
            bo = b * CH + g * LANES
            sv = sbuf[pl.ds(bo, LANES)]
            n0v = nbuf[pl.ds(bo, LANES)]
            for d in range(D):
                t0d = t0a[d] if d < LANES else t0b[d - LANES]
                v = obuf[D + d, pl.ds(bo, LANES)]
                obuf[D + d, pl.ds(bo, LANES)] = (v - n0v * t0d) * sv

    fire(0, base0)

    def pair_body(k, carry):
        c0 = 2 * k
        fire(1, base0 + (c0 + 1) * CH)
        drain_gathers(0)

        @pl.when(k > 0)
        def _():
            out_copy(0, base0 + (c0 - 2) * CH).wait()

        compute(0, base0 + c0 * CH)
        out_copy(0, base0 + c0 * CH).start()

        @pl.when(c0 + 2 < NCH)
        def _():
            fire(0, base0 + (c0 + 2) * CH)

        drain_gathers(1)

        @pl.when(k > 0)
        def _():
            out_copy(1, base0 + (c0 - 1) * CH).wait()

        compute(1, base0 + (c0 + 1) * CH)
        out_copy(1, base0 + (c0 + 1) * CH).start()
        return carry

    lax.fori_loop(0, NCH // 2, pair_body, 0, unroll=False)
    out_copy(0, base0 + (NCH - 2) * CH).wait()
    out_copy(1, base0 + (NCH - 1) * CH).wait()


@jax.jit
def _run(title_ids, tok_flat, ttf, token_table):
    mesh = plsc.VectorSubcoreMesh(
        core_axis_name="c", subcore_axis_name="s",
        num_cores=NC, num_subcores=NS)
    f = pl.kernel(
        _body,
        out_type=jax.ShapeDtypeStruct((2 * D, B), jnp.float32),
        mesh=mesh,
        compiler_params=pltpu.CompilerParams(
            needs_layout_passes=False, use_tc_tiling_on_sc=False),
        scratch_types=[
            pltpu.VMEM((2 * CH,), jnp.int32),
            pltpu.VMEM((2 * TW,), jnp.int32),
            pltpu.VMEM((2 * CH * L,), jnp.int32),
            pltpu.VMEM((2 * TW,), jnp.float32),
            pltpu.VMEM((2 * CH * L, D), jnp.float32),
            pltpu.VMEM((2 * D, 2 * CH), jnp.float32),
            pltpu.VMEM((2 * CH,), jnp.float32),
            pltpu.VMEM((2 * CH,), jnp.float32),
            pltpu.VMEM((1, D), jnp.float32),
            pltpu.SemaphoreType.DMA,
            pltpu.SemaphoreType.DMA,
            pltpu.SemaphoreType.DMA,
            pltpu.SemaphoreType.DMA,
        ],
    )
    return f(title_ids, tok_flat, ttf, token_table)


def kernel(title_ids, token_ids, title_table, token_table):
    tok_flat = token_ids.reshape(B * L)
    ttf = title_table.T.reshape(D * NT)
    out_t = _run(title_ids, tok_flat, ttf, token_table)
    return out_t.T


# speedup vs baseline: 1.7651x; 1.1424x over previous
import functools

import jax
import jax.numpy as jnp
from jax import lax
from jax.experimental import pallas as pl
from jax.experimental.pallas import tpu as pltpu
from jax.experimental.pallas import tpu_sc as plsc

NC = 2
NS = 16
LANES = 16
NW = NC * NS

B = 16384
L = 20
D = 32
CH = 64
ROWS_PER_W = B // NW
NCH = ROWS_PER_W // CH
GSTEP = 128
NGS = CH * L // GSTEP
NT = 100001
TW = CH * D
NTS = TW // GSTEP


def _body(tid_hbm, kid_hbm, ttf_hbm, ktab_hbm, out_hbm,
          tidx, tgidx, kidx, tbufT, kbuf, obuf, sbuf, nbuf, t0buf,
          sg0, sg1, so0, so1):
    wid = lax.axis_index("s") * NC + lax.axis_index("c")
    base0 = wid * ROWS_PER_W
    sem_g = (sg0, sg1)
    sem_o = (so0, so1)

    pltpu.sync_copy(ktab_hbm.at[pl.ds(0, 1)], t0buf)
    t0a = t0buf[0, pl.ds(0, LANES)]
    t0b = t0buf[0, pl.ds(LANES, LANES)]
    lanes = lax.iota(jnp.int32, 16)

    def fire(b, base):
        pltpu.sync_copy(tid_hbm.at[pl.ds(base, CH)], tidx.at[pl.ds(b * CH, CH)])
        pltpu.sync_copy(kid_hbm.at[:, pl.ds(base, CH)],
                        kidx.at[:, pl.ds(b * CH, CH)])
        for g in range(CH // LANES):
            v = tidx[pl.ds(b * CH + g * LANES, LANES)]
            for d in range(D):
                tgidx[pl.ds(b * TW + d * CH + g * LANES, LANES)] = v + d * NT
        for p in range(NTS):
            o = b * TW + p * GSTEP
            pltpu.async_copy(ttf_hbm.at[tgidx.at[pl.ds(o, GSTEP)]],
                             tbufT.at[pl.ds(o, GSTEP)], sem_g[b])
        for j in range(L):
            pltpu.async_copy(ktab_hbm.at[kidx.at[j, pl.ds(b * CH, CH)]],
                             kbuf.at[pl.ds(b * CH * L + j * CH, CH)],
                             sem_g[b])

    def drain_gathers(b):
        for p in range(NTS):
            o = b * TW + p * GSTEP
            pltpu.make_async_copy(ttf_hbm.at[tgidx.at[pl.ds(o, GSTEP)]],
                                  tbufT.at[pl.ds(o, GSTEP)], sem_g[b]).wait()
        for j in range(L):
            pltpu.make_async_copy(ktab_hbm.at[kidx.at[j, pl.ds(b * CH, CH)]],
                                  kbuf.at[pl.ds(b * CH * L + j * CH, CH)],
                                  sem_g[b]).wait()

    def out_copy(b, base):
        return pltpu.make_async_copy(
            obuf.at[:, pl.ds(b * CH, CH)],
            out_hbm.at[:, pl.ds(base, CH)], sem_o[b])

    def compute(b, base):
        kb = b * CH * L
        for g in range(CH // LANES):
            acc = jnp.zeros((LANES,), jnp.int32)
            for j in range(L):
                ids = kidx[j, pl.ds(b * CH + g * LANES, LANES)]
                acc = acc + jnp.where(ids != 0, 1, 0)
            nf = acc.astype(jnp.float32)
            bo = b * CH + g * LANES
            sbuf[pl.ds(bo, LANES)] = 1.0 / jnp.maximum(nf, 1.0)
            nbuf[pl.ds(bo, LANES)] = jnp.float32(L) - nf

        for g in range(CH // LANES):
            bo = b * CH + g * LANES
            for d in range(D):
                obuf[d, pl.ds(bo, LANES)] = tbufT[pl.ds(b * TW + d * CH
                                                        + g * LANES, LANES)]

        def row_body(i, carry):
            r0 = kb + i
            acc0 = kbuf[r0, pl.ds(0, LANES)]
            acc1 = kbuf[r0, pl.ds(LANES, LANES)]
            for j in range(1, L):
                acc0 = acc0 + kbuf[r0 + j * CH, pl.ds(0, LANES)]
                acc1 = acc1 + kbuf[r0 + j * CH, pl.ds(LANES, LANES)]
            col = lanes * 0 + (b * CH + i)
            plsc.store_scatter(obuf, [2 * LANES + lanes, col], acc0)
            plsc.store_scatter(obuf, [3 * LANES + lanes, col], acc1)
            return carry

        lax.fori_loop(0, CH, row_body, 0, unroll=False)

        for g in range(CH // LANES):
            bo = b * CH + g * LANES
            sv = sbuf[pl.ds(bo, LANES)]
            n0v = nbuf[pl.ds(bo, LANES)]
            for d in range(D):
                t0d = t0a[d] if d < LANES else t0b[d - LANES]
                v = obuf[D + d, pl.ds(bo, LANES)]
                obuf[D + d, pl.ds(bo, LANES)] = (v - n0v * t0d) * sv

    fire(0, base0)

    def pair_body(k, carry):
        c0 = 2 * k
        fire(1, base0 + (c0 + 1) * CH)
        drain_gathers(0)

        @pl.when(k > 0)
        def _():
            out_copy(0, base0 + (c0 - 2) * CH).wait()

        compute(0, base0 + c0 * CH)
        out_copy(0, base0 + c0 * CH).start()

        @pl.when(c0 + 2 < NCH)
        def _():
            fire(0, base0 + (c0 + 2) * CH)

        drain_gathers(1)

        @pl.when(k > 0)
        def _():
            out_copy(1, base0 + (c0 - 1) * CH).wait()

        compute(1, base0 + (c0 + 1) * CH)
        out_copy(1, base0 + (c0 + 1) * CH).start()
        return carry

    lax.fori_loop(0, NCH // 2, pair_body, 0, unroll=False)
    out_copy(0, base0 + (NCH - 2) * CH).wait()
    out_copy(1, base0 + (NCH - 1) * CH).wait()


@jax.jit
def _run(title_ids, tok_flat, ttf, token_table):
    mesh = plsc.VectorSubcoreMesh(
        core_axis_name="c", subcore_axis_name="s",
        num_cores=NC, num_subcores=NS)
    f = pl.kernel(
        _body,
        out_type=jax.ShapeDtypeStruct((2 * D, B), jnp.float32),
        mesh=mesh,
        compiler_params=pltpu.CompilerParams(
            needs_layout_passes=False, use_tc_tiling_on_sc=False),
        scratch_types=[
            pltpu.VMEM((2 * CH,), jnp.int32),
            pltpu.VMEM((2 * TW,), jnp.int32),
            pltpu.VMEM((L, 2 * CH), jnp.int32),
            pltpu.VMEM((2 * TW,), jnp.float32),
            pltpu.VMEM((2 * CH * L, D), jnp.float32),
            pltpu.VMEM((2 * D, 2 * CH), jnp.float32),
            pltpu.VMEM((2 * CH,), jnp.float32),
            pltpu.VMEM((2 * CH,), jnp.float32),
            pltpu.VMEM((1, D), jnp.float32),
            pltpu.SemaphoreType.DMA,
            pltpu.SemaphoreType.DMA,
            pltpu.SemaphoreType.DMA,
            pltpu.SemaphoreType.DMA,
        ],
    )
    return f(title_ids, tok_flat, ttf, token_table)


def kernel(title_ids, token_ids, title_table, token_table):
    tok_t = token_ids.T
    ttf = title_table.T.reshape(D * NT)
    out_t = _run(title_ids, tok_t, ttf, token_table)
    return out_t.T
